# Initial kernel scaffold; baseline (speedup 1.0000x reference)
import jax
import jax.numpy as jnp
from jax.experimental import pallas as pl


def kernel(input_ids, table, ln_weight, conv_weight):
    raise NotImplementedError("write your pallas kernel here")


# speedup vs baseline: 1.0420x; 1.0420x over previous
import functools

import jax
import jax.numpy as jnp
from jax import lax
from jax.experimental import pallas as pl
from jax.experimental.pallas import tpu as pltpu
from jax.experimental.pallas import tpu_sc as plsc

VOCAB = 8192
IN_CH = 768
OUT_CH = 768
EPS = 1e-06

_NC = 2
_NS = 16
_NW = _NC * _NS


def _make_gather(B: int, D: int, chunk: int):
    assert B % (8 * _NW) == 0
    b_per_w = B // _NW
    assert b_per_w % chunk == 0
    n_chunks = b_per_w // chunk
    mesh = plsc.VectorSubcoreMesh(core_axis_name="c", subcore_axis_name="s")

    @functools.partial(
        pl.kernel,
        mesh=mesh,
        out_type=jax.ShapeDtypeStruct((B, D), jnp.float32),
        scratch_types=[
            pltpu.VMEM((b_per_w,), jnp.int32),
            pltpu.VMEM((chunk, D), jnp.float32),
            pltpu.VMEM((chunk, D), jnp.float32),
            pltpu.SemaphoreType.DMA,
            pltpu.SemaphoreType.DMA,
        ],
    )
    def gather_kernel(idx_hbm, table_hbm, out_hbm, idx_v, rows0, rows1, sem0, sem1):
        wid = lax.axis_index("s") * _NC + lax.axis_index("c")
        base = wid * b_per_w
        pltpu.sync_copy(idx_hbm.at[pl.ds(base, b_per_w)], idx_v)
        rows = (rows0, rows1)
        sems = (sem0, sem1)
        copies = [None, None]
        copies[0] = pltpu.async_copy(
            table_hbm.at[idx_v.at[pl.ds(0, chunk)]], rows[0], sems[0])
        for ci in range(n_chunks):
            cur = ci % 2
            nxt = (ci + 1) % 2
            if ci + 1 < n_chunks:
                copies[nxt] = pltpu.async_copy(
                    table_hbm.at[idx_v.at[pl.ds((ci + 1) * chunk, chunk)]],
                    rows[nxt], sems[nxt])
            copies[cur].wait()
            pltpu.sync_copy(rows[cur], out_hbm.at[pl.ds(base + ci * chunk, chunk)])

    return gather_kernel


def _norm_matmul_body(emb_ref, w_ref, ln_ref, out_ref):
    x = emb_ref[0]
    ssq = jnp.sum(x * x, axis=1, keepdims=True)
    scale = lax.rsqrt(ssq * (1.0 / IN_CH) + EPS)
    xs = (x * scale) * ln_ref[0]
    out_ref[0] = lax.dot_general(
        w_ref[...], xs,
        dimension_numbers=(((1,), (1,)), ((), ())),
        preferred_element_type=jnp.float32,
    )


def kernel(input_ids, table, ln_weight, conv_weight):
    Bt, H, W = input_ids.shape
    B = Bt * H * W
    ids_flat = input_ids.reshape(B).astype(jnp.int32)

    emb = _make_gather(B, IN_CH, chunk=64)(ids_flat, table)

    HW = H * W
    emb3 = emb.reshape(Bt, HW, IN_CH)
    ln2 = ln_weight.reshape(1, IN_CH)

    out = pl.pallas_call(
        _norm_matmul_body,
        grid=(Bt,),
        in_specs=[
            pl.BlockSpec((1, HW, IN_CH), lambda b: (b, 0, 0)),
            pl.BlockSpec((OUT_CH, IN_CH), lambda b: (0, 0)),
            pl.BlockSpec((1, IN_CH), lambda b: (0, 0)),
        ],
        out_specs=pl.BlockSpec((1, OUT_CH, HW), lambda b: (b, 0, 0)),
        out_shape=jax.ShapeDtypeStruct((Bt, OUT_CH, HW), jnp.float32),
    )(emb3, conv_weight, ln2)

    return out.reshape(Bt, OUT_CH, H, W)
